# baseline (speedup 1.0000x reference)
import functools

import jax
import jax.numpy as jnp
from jax import lax
from jax.experimental import pallas as pl
from jax.experimental.pallas import tpu as pltpu
from jax.experimental.pallas import tpu_sc as plsc

_HIDDEN = 2048
_BATCH = 16384
_NC = 2
_NS = 16
_NW = _NC * _NS
_ROWS_PER_W = _BATCH // _NW
_REP = 32
_N_DMA = _ROWS_PER_W // _REP

_mesh = plsc.VectorSubcoreMesh(core_axis_name="c", subcore_axis_name="s")


@functools.partial(
    pl.kernel,
    out_type=jax.ShapeDtypeStruct((_BATCH, _HIDDEN), jnp.float32),
    mesh=_mesh,
    scratch_types=[
        pltpu.VMEM((_REP, _HIDDEN), jnp.float32),
        pltpu.VMEM_SHARED((_ROWS_PER_W, _HIDDEN), jnp.float32),
        pltpu.SemaphoreType.DMA,
    ],
)
def _broadcast_row(w_hbm, out_hbm, buf, shared, sem):
    c = lax.axis_index("c")
    s = lax.axis_index("s")
    fills = [
        pltpu.async_copy(w_hbm, buf.at[pl.ds(i, 1)], sem) for i in range(_REP)
    ]
    for f in fills:
        f.wait()
    pltpu.sync_copy(buf, shared.at[pl.ds(s * _REP, _REP)])
    plsc.subcore_barrier()
    base = (c * _NS + s) * _ROWS_PER_W
    pltpu.sync_copy(shared, out_hbm.at[pl.ds(base, _ROWS_PER_W)])


def kernel(x, embedding_weight):
    del x
    return _broadcast_row(embedding_weight)


# speedup vs baseline: 1.2731x; 1.2731x over previous
import functools

import jax
import jax.numpy as jnp
from jax import lax
from jax.experimental import pallas as pl
from jax.experimental.pallas import tpu as pltpu
from jax.experimental.pallas import tpu_sc as plsc

_HIDDEN = 2048
_BATCH = 16384
_NC = 2
_NS = 16
_NW = _NC * _NS
_ROWS_PER_W = _BATCH // _NW
_REP = 32
_N_DMA = _ROWS_PER_W // _REP

_mesh = plsc.VectorSubcoreMesh(core_axis_name="c", subcore_axis_name="s")


@functools.partial(
    pl.kernel,
    out_type=jax.ShapeDtypeStruct((_BATCH, _HIDDEN), jnp.float32),
    mesh=_mesh,
    scratch_types=[
        pltpu.VMEM((_REP, _HIDDEN), jnp.float32),
        pltpu.SemaphoreType.DMA,
    ],
)
def _broadcast_row(w_hbm, out_hbm, buf, sem):
    wid = lax.axis_index("s") * _NC + lax.axis_index("c")
    base = wid * _ROWS_PER_W
    fills = [
        pltpu.async_copy(w_hbm, buf.at[pl.ds(i, 1)], sem) for i in range(_REP)
    ]
    for f in fills:
        f.wait()
    copies = [
        pltpu.async_copy(buf, out_hbm.at[pl.ds(base + i * _REP, _REP)], sem)
        for i in range(_N_DMA)
    ]
    for c in copies:
        c.wait()


def kernel(x, embedding_weight):
    del x
    return _broadcast_row(embedding_weight)
